# Initial kernel scaffold; baseline (speedup 1.0000x reference)
import jax
import jax.numpy as jnp
from jax.experimental import pallas as pl


def kernel(H_0, Z, block_id, batch_id, edges, edge_attr):
    raise NotImplementedError("write your pallas kernel here")


# speedup vs baseline: 3.4163x; 3.4163x over previous
import functools

import jax
import jax.numpy as jnp
from jax import lax
from jax.experimental import pallas as pl
from jax.experimental.pallas import tpu as pltpu
from jax.experimental.pallas import tpu_sc as plsc

N_NODES = 10000
D_FEAT = 128
N_BLOCKS = 500
N_BLOCKS_PAD = 512
N_GRAPHS = 16

TILE = 16
TILES_PER_CHUNK = 5
CHUNK = TILE * TILES_PER_CHUNK
N_CHUNKS = N_NODES // CHUNK
CHUNKS_PER_SUBCORE = -(-N_CHUNKS // 16)

D_HALF = D_FEAT // 2


def _sc_body(h0_hbm, bid_hbm, gid_hbm, zeros_hbm, blk_out, gr_out,
             rows_v, idxb_v, idxg_v, blk_acc, gr_acc):
    c = lax.axis_index("c")
    s = lax.axis_index("s")
    col0 = c * D_HALF

    pltpu.sync_copy(zeros_hbm, blk_acc.at[pl.ds(s * 32, 32)])

    @pl.when(s == 0)
    def _():
        pltpu.sync_copy(zeros_hbm.at[pl.ds(0, N_GRAPHS)], gr_acc)

    plsc.subcore_barrier()

    for k in range(CHUNKS_PER_SUBCORE):
        ch = s * CHUNKS_PER_SUBCORE + k

        @pl.when(ch < N_CHUNKS)
        def _():
            r0 = ch * CHUNK
            pltpu.sync_copy(h0_hbm.at[pl.ds(r0, CHUNK), pl.ds(col0, D_HALF)],
                            rows_v)
            pltpu.sync_copy(bid_hbm.at[pl.ds(ch * TILES_PER_CHUNK,
                                             TILES_PER_CHUNK)], idxb_v)
            pltpu.sync_copy(gid_hbm.at[pl.ds(ch * TILES_PER_CHUNK,
                                             TILES_PER_CHUNK)], idxg_v)
            for j in range(TILES_PER_CHUNK):
                rows = rows_v.at[pl.ds(TILE * j, TILE)]
                pltpu.sync_copy(rows, blk_acc.at[idxb_v.at[j]], add=True)
                pltpu.sync_copy(rows, gr_acc.at[idxg_v.at[j]], add=True)

    plsc.subcore_barrier()

    @pl.when(s < 15)
    def _():
        pltpu.sync_copy(blk_acc.at[pl.ds(s * 32, 32)],
                        blk_out.at[pl.ds(s * 32, 32), pl.ds(col0, D_HALF)])

    @pl.when(s == 15)
    def _():
        pltpu.sync_copy(blk_acc.at[pl.ds(480, N_BLOCKS - 480)],
                        blk_out.at[pl.ds(480, N_BLOCKS - 480),
                                   pl.ds(col0, D_HALF)])

    @pl.when(s == 0)
    def _():
        pltpu.sync_copy(gr_acc, gr_out.at[:, pl.ds(col0, D_HALF)])


@jax.jit
def _segsum_sc(h0, bid2d, gid2d, zeros_init):
    mesh = plsc.VectorSubcoreMesh(core_axis_name="c", subcore_axis_name="s")
    f = pl.kernel(
        _sc_body,
        out_type=(
            jax.ShapeDtypeStruct((N_BLOCKS, D_FEAT), jnp.float32),
            jax.ShapeDtypeStruct((N_GRAPHS, D_FEAT), jnp.float32),
        ),
        mesh=mesh,
        compiler_params=pltpu.CompilerParams(use_tc_tiling_on_sc=False),
        scratch_types=[
            pltpu.VMEM((CHUNK, D_HALF), jnp.float32),
            pltpu.VMEM((TILES_PER_CHUNK, TILE), jnp.int32),
            pltpu.VMEM((TILES_PER_CHUNK, TILE), jnp.int32),
            pltpu.VMEM_SHARED((N_BLOCKS_PAD, D_HALF), jnp.float32),
            pltpu.VMEM_SHARED((N_GRAPHS, D_HALF), jnp.float32),
        ],
    )
    return f(h0, bid2d, gid2d, zeros_init)


def kernel(H_0, Z, block_id, batch_id, edges, edge_attr):
    bid2d = block_id.astype(jnp.int32).reshape(N_NODES // TILE, TILE)
    gid2d = batch_id.astype(jnp.int32).reshape(N_NODES // TILE, TILE)
    zeros_init = jnp.zeros((32, D_HALF), jnp.float32)
    block_repr, graph_repr = _segsum_sc(H_0, bid2d, gid2d, zeros_init)
    return (H_0, block_repr, graph_repr, Z)


# speedup vs baseline: 4.7059x; 1.3775x over previous
import jax
import jax.numpy as jnp
from jax import lax
from jax.experimental import pallas as pl
from jax.experimental.pallas import tpu as pltpu
from jax.experimental.pallas import tpu_sc as plsc

N_NODES = 10000
D_FEAT = 128
N_BLOCKS = 500
N_BLOCKS_PAD = 512
N_GRAPHS = 16

TILE = 16
N_TILES = N_NODES // TILE
TILES_PER_SUB = 40
ROWS_PER_SUB = TILES_PER_SUB * TILE
TAIL_ROWS = 16 * ROWS_PER_SUB - N_NODES

D_HALF = D_FEAT // 2


def _sc_body(h0_hbm, bid_hbm, gid_hbm, zeros_hbm, blk_out, gr_out,
             rows_v, idxb_v, idxg_v, blk_acc, gr_acc, sem_h, sem_i):
    c = lax.axis_index("c")
    s = lax.axis_index("s")
    col0 = c * D_HALF
    r0 = s * ROWS_PER_SUB

    pltpu.async_copy(bid_hbm.at[pl.ds(s * TILES_PER_SUB, TILES_PER_SUB)],
                     idxb_v, sem_i)
    pltpu.async_copy(gid_hbm.at[pl.ds(s * TILES_PER_SUB, TILES_PER_SUB)],
                     idxg_v, sem_i)

    @pl.when(s < 15)
    def _():
        pltpu.async_copy(h0_hbm.at[pl.ds(r0, ROWS_PER_SUB),
                                   pl.ds(col0, D_HALF)], rows_v, sem_h)

    @pl.when(s == 15)
    def _():
        valid = ROWS_PER_SUB - TAIL_ROWS
        pltpu.async_copy(h0_hbm.at[pl.ds(15 * ROWS_PER_SUB, valid),
                                   pl.ds(col0, D_HALF)],
                         rows_v.at[pl.ds(0, valid)], sem_h)
        pltpu.async_copy(zeros_hbm, rows_v.at[pl.ds(valid, TAIL_ROWS)], sem_h)

    pltpu.sync_copy(zeros_hbm.at[pl.ds(0, 32)],
                    blk_acc.at[pl.ds(s * 32, 32)])

    @pl.when(s == 0)
    def _():
        pltpu.sync_copy(zeros_hbm.at[pl.ds(0, N_GRAPHS)], gr_acc)

    plsc.subcore_barrier()

    pltpu.make_async_copy(bid_hbm.at[pl.ds(s * TILES_PER_SUB,
                                           TILES_PER_SUB)],
                          idxb_v, sem_i).wait()
    pltpu.make_async_copy(gid_hbm.at[pl.ds(s * TILES_PER_SUB,
                                           TILES_PER_SUB)],
                          idxg_v, sem_i).wait()

    @pl.when(s < 15)
    def _():
        pltpu.make_async_copy(h0_hbm.at[pl.ds(r0, ROWS_PER_SUB),
                                        pl.ds(col0, D_HALF)],
                              rows_v, sem_h).wait()

    @pl.when(s == 15)
    def _():
        valid = ROWS_PER_SUB - TAIL_ROWS
        pltpu.make_async_copy(h0_hbm.at[pl.ds(15 * ROWS_PER_SUB, valid),
                                        pl.ds(col0, D_HALF)],
                              rows_v.at[pl.ds(0, valid)], sem_h).wait()
        pltpu.make_async_copy(zeros_hbm, rows_v.at[pl.ds(valid, TAIL_ROWS)],
                              sem_h).wait()

    descs = []
    for t in range(TILES_PER_SUB):
        rows = rows_v.at[pl.ds(TILE * t, TILE)]
        descs.append(pltpu.async_copy(rows, blk_acc.at[idxb_v.at[t]],
                                      sem_h, add=True))
        descs.append(pltpu.async_copy(rows, gr_acc.at[idxg_v.at[t]],
                                      sem_h, add=True))
    for d in descs:
        d.wait()

    plsc.subcore_barrier()

    @pl.when(s < 15)
    def _():
        pltpu.sync_copy(blk_acc.at[pl.ds(s * 32, 32)],
                        blk_out.at[pl.ds(s * 32, 32), pl.ds(col0, D_HALF)])

    @pl.when(s == 15)
    def _():
        pltpu.sync_copy(blk_acc.at[pl.ds(480, N_BLOCKS - 480)],
                        blk_out.at[pl.ds(480, N_BLOCKS - 480),
                                   pl.ds(col0, D_HALF)])

    @pl.when(s == 0)
    def _():
        pltpu.sync_copy(gr_acc, gr_out.at[:, pl.ds(col0, D_HALF)])


@jax.jit
def _segsum_sc(h0, bid2d, gid2d, zeros_init):
    mesh = plsc.VectorSubcoreMesh(core_axis_name="c", subcore_axis_name="s")
    f = pl.kernel(
        _sc_body,
        out_type=(
            jax.ShapeDtypeStruct((N_BLOCKS, D_FEAT), jnp.float32),
            jax.ShapeDtypeStruct((N_GRAPHS, D_FEAT), jnp.float32),
        ),
        mesh=mesh,
        compiler_params=pltpu.CompilerParams(use_tc_tiling_on_sc=False),
        scratch_types=[
            pltpu.VMEM((ROWS_PER_SUB, D_HALF), jnp.float32),
            pltpu.VMEM((TILES_PER_SUB, TILE), jnp.int32),
            pltpu.VMEM((TILES_PER_SUB, TILE), jnp.int32),
            pltpu.VMEM_SHARED((N_BLOCKS_PAD, D_HALF), jnp.float32),
            pltpu.VMEM_SHARED((N_GRAPHS, D_HALF), jnp.float32),
            pltpu.SemaphoreType.DMA,
            pltpu.SemaphoreType.DMA,
        ],
    )
    return f(h0, bid2d, gid2d, zeros_init)


def kernel(H_0, Z, block_id, batch_id, edges, edge_attr):
    bid2d = jnp.pad(block_id.astype(jnp.int32).reshape(N_TILES, TILE),
                    ((0, 16 * TILES_PER_SUB - N_TILES), (0, 0)))
    gid2d = jnp.pad(batch_id.astype(jnp.int32).reshape(N_TILES, TILE),
                    ((0, 16 * TILES_PER_SUB - N_TILES), (0, 0)))
    zeros_init = jnp.zeros((TAIL_ROWS, D_HALF), jnp.float32)
    block_repr, graph_repr = _segsum_sc(H_0, bid2d, gid2d, zeros_init)
    return (H_0, block_repr, graph_repr, Z)


# speedup vs baseline: 5.0681x; 1.0770x over previous
import jax
import jax.numpy as jnp
from jax import lax
from jax.experimental import pallas as pl
from jax.experimental.pallas import tpu as pltpu
from jax.experimental.pallas import tpu_sc as plsc

N_NODES = 10000
D_FEAT = 128
N_BLOCKS = 500
N_BLOCKS_PAD = 512
N_GRAPHS = 16

IDXW = 128
GROUPS_PER_SUB = 5
ROWS_PER_SUB = IDXW * GROUPS_PER_SUB
N_NODES_PAD = 16 * ROWS_PER_SUB
TAIL_ROWS = N_NODES_PAD - N_NODES

D_HALF = D_FEAT // 2


def _sc_body(h0_hbm, bid_hbm, gid_hbm, zeros_hbm, blk_out, gr_out,
             rows_v, idxb_v, idxg_v, blk_acc, gr_acc, sem_h, sem_i):
    c = lax.axis_index("c")
    s = lax.axis_index("s")
    col0 = c * D_HALF
    r0 = s * ROWS_PER_SUB

    pltpu.async_copy(bid_hbm.at[pl.ds(s * GROUPS_PER_SUB, GROUPS_PER_SUB)],
                     idxb_v, sem_i)
    pltpu.async_copy(gid_hbm.at[pl.ds(s * GROUPS_PER_SUB, GROUPS_PER_SUB)],
                     idxg_v, sem_i)

    @pl.when(s < 15)
    def _():
        pltpu.async_copy(h0_hbm.at[pl.ds(r0, ROWS_PER_SUB),
                                   pl.ds(col0, D_HALF)], rows_v, sem_h)

    @pl.when(s == 15)
    def _():
        valid = ROWS_PER_SUB - TAIL_ROWS
        pltpu.async_copy(h0_hbm.at[pl.ds(15 * ROWS_PER_SUB, valid),
                                   pl.ds(col0, D_HALF)],
                         rows_v.at[pl.ds(0, valid)], sem_h)
        pltpu.async_copy(zeros_hbm, rows_v.at[pl.ds(valid, TAIL_ROWS)], sem_h)

    pltpu.sync_copy(zeros_hbm.at[pl.ds(0, 32)],
                    blk_acc.at[pl.ds(s * 32, 32)])

    @pl.when(s == 0)
    def _():
        pltpu.sync_copy(zeros_hbm.at[pl.ds(0, N_GRAPHS)], gr_acc)

    plsc.subcore_barrier()

    pltpu.make_async_copy(bid_hbm.at[pl.ds(s * GROUPS_PER_SUB,
                                           GROUPS_PER_SUB)],
                          idxb_v, sem_i).wait()
    pltpu.make_async_copy(gid_hbm.at[pl.ds(s * GROUPS_PER_SUB,
                                           GROUPS_PER_SUB)],
                          idxg_v, sem_i).wait()

    @pl.when(s < 15)
    def _():
        pltpu.make_async_copy(h0_hbm.at[pl.ds(r0, ROWS_PER_SUB),
                                        pl.ds(col0, D_HALF)],
                              rows_v, sem_h).wait()

    @pl.when(s == 15)
    def _():
        valid = ROWS_PER_SUB - TAIL_ROWS
        pltpu.make_async_copy(h0_hbm.at[pl.ds(15 * ROWS_PER_SUB, valid),
                                        pl.ds(col0, D_HALF)],
                              rows_v.at[pl.ds(0, valid)], sem_h).wait()
        pltpu.make_async_copy(zeros_hbm, rows_v.at[pl.ds(valid, TAIL_ROWS)],
                              sem_h).wait()

    descs = []
    for j in range(GROUPS_PER_SUB):
        rows = rows_v.at[pl.ds(IDXW * j, IDXW)]
        descs.append(pltpu.async_copy(rows, blk_acc.at[idxb_v.at[j]],
                                      sem_h, add=True))
        descs.append(pltpu.async_copy(rows, gr_acc.at[idxg_v.at[j]],
                                      sem_h, add=True))
    for d in descs:
        d.wait()

    plsc.subcore_barrier()

    @pl.when(s < 15)
    def _():
        pltpu.sync_copy(blk_acc.at[pl.ds(s * 32, 32)],
                        blk_out.at[pl.ds(s * 32, 32), pl.ds(col0, D_HALF)])

    @pl.when(s == 15)
    def _():
        pltpu.sync_copy(blk_acc.at[pl.ds(480, N_BLOCKS - 480)],
                        blk_out.at[pl.ds(480, N_BLOCKS - 480),
                                   pl.ds(col0, D_HALF)])

    @pl.when(s == 0)
    def _():
        pltpu.sync_copy(gr_acc, gr_out.at[:, pl.ds(col0, D_HALF)])


@jax.jit
def _segsum_sc(h0, bid2d, gid2d, zeros_init):
    mesh = plsc.VectorSubcoreMesh(core_axis_name="c", subcore_axis_name="s")
    f = pl.kernel(
        _sc_body,
        out_type=(
            jax.ShapeDtypeStruct((N_BLOCKS, D_FEAT), jnp.float32),
            jax.ShapeDtypeStruct((N_GRAPHS, D_FEAT), jnp.float32),
        ),
        mesh=mesh,
        compiler_params=pltpu.CompilerParams(use_tc_tiling_on_sc=False),
        scratch_types=[
            pltpu.VMEM((ROWS_PER_SUB, D_HALF), jnp.float32),
            pltpu.VMEM((GROUPS_PER_SUB, IDXW), jnp.int32),
            pltpu.VMEM((GROUPS_PER_SUB, IDXW), jnp.int32),
            pltpu.VMEM_SHARED((N_BLOCKS_PAD, D_HALF), jnp.float32),
            pltpu.VMEM_SHARED((N_GRAPHS, D_HALF), jnp.float32),
            pltpu.SemaphoreType.DMA,
            pltpu.SemaphoreType.DMA,
        ],
    )
    return f(h0, bid2d, gid2d, zeros_init)


def kernel(H_0, Z, block_id, batch_id, edges, edge_attr):
    bid2d = jnp.pad(block_id.astype(jnp.int32),
                    (0, N_NODES_PAD - N_NODES)).reshape(-1, IDXW)
    gid2d = jnp.pad(batch_id.astype(jnp.int32),
                    (0, N_NODES_PAD - N_NODES)).reshape(-1, IDXW)
    zeros_init = jnp.zeros((TAIL_ROWS, D_HALF), jnp.float32)
    block_repr, graph_repr = _segsum_sc(H_0, bid2d, gid2d, zeros_init)
    return (H_0, block_repr, graph_repr, Z)


# speedup vs baseline: 5.3109x; 1.0479x over previous
import jax
import jax.numpy as jnp
from jax import lax
from jax.experimental import pallas as pl
from jax.experimental.pallas import tpu as pltpu
from jax.experimental.pallas import tpu_sc as plsc

N_NODES = 10000
D_FEAT = 128
N_BLOCKS = 500
N_BLOCKS_PAD = 512
N_GRAPHS = 16
N_GRAPHS_PAD = 32

IDXW = 128
GROUPS_PER_SUB = 5
ROWS_PER_SUB = IDXW * GROUPS_PER_SUB
VALID_LAST = N_NODES - 15 * ROWS_PER_SUB

D_HALF = D_FEAT // 2


def _sc_body(h0_hbm, bid_hbm, gid_hbm, blk_out, gr_out,
             rows_v, idxb_v, idxg_v, zero_v, blk_acc, gr_acc,
             sem_h, sem_i):
    c = lax.axis_index("c")
    s = lax.axis_index("s")
    col0 = c * D_HALF
    r0 = s * ROWS_PER_SUB

    @pl.when(s < 15)
    def _():
        pltpu.async_copy(h0_hbm.at[pl.ds(r0, ROWS_PER_SUB),
                                   pl.ds(col0, D_HALF)], rows_v, sem_h)
        for j in range(GROUPS_PER_SUB):
            src = pl.ds(r0 + IDXW * j, IDXW)
            pltpu.async_copy(bid_hbm.at[src], idxb_v.at[j], sem_i)
            pltpu.async_copy(gid_hbm.at[src], idxg_v.at[j], sem_i)

    @pl.when(s == 15)
    def _():
        pltpu.async_copy(h0_hbm.at[pl.ds(r0, VALID_LAST),
                                   pl.ds(col0, D_HALF)],
                         rows_v.at[pl.ds(0, VALID_LAST)], sem_h)
        for j in range(3):
            src = pl.ds(r0 + IDXW * j, IDXW)
            pltpu.async_copy(bid_hbm.at[src], idxb_v.at[j], sem_i)
            pltpu.async_copy(gid_hbm.at[src], idxg_v.at[j], sem_i)
        tail = pl.ds(r0 + IDXW * 3, VALID_LAST - IDXW * 3)
        pltpu.async_copy(bid_hbm.at[tail], idxb_v.at[3, pl.ds(0, 16)], sem_i)
        pltpu.async_copy(gid_hbm.at[tail], idxg_v.at[3, pl.ds(0, 16)], sem_i)

    z16 = jnp.zeros((16,), jnp.float32)
    for i in range(32):
        for j in range(4):
            zero_v[i, pl.ds(16 * j, 16)] = z16

    @pl.when(s == 15)
    def _():
        dump_b = jnp.full((16,), N_BLOCKS_PAD - 1, jnp.int32)
        dump_g = jnp.full((16,), N_GRAPHS_PAD - 1, jnp.int32)
        for k in range(1, 8):
            idxb_v[3, pl.ds(16 * k, 16)] = dump_b
            idxg_v[3, pl.ds(16 * k, 16)] = dump_g
        for k in range(8):
            idxb_v[4, pl.ds(16 * k, 16)] = dump_b
            idxg_v[4, pl.ds(16 * k, 16)] = dump_g

    pltpu.sync_copy(zero_v, blk_acc.at[pl.ds(s * 32, 32)])

    @pl.when(s == 0)
    def _():
        pltpu.sync_copy(zero_v, gr_acc)

    plsc.subcore_barrier()

    @pl.when(s < 15)
    def _():
        pltpu.make_async_copy(h0_hbm.at[pl.ds(r0, ROWS_PER_SUB),
                                        pl.ds(col0, D_HALF)],
                              rows_v, sem_h).wait()
        for j in range(GROUPS_PER_SUB):
            src = pl.ds(r0 + IDXW * j, IDXW)
            pltpu.make_async_copy(bid_hbm.at[src], idxb_v.at[j], sem_i).wait()
            pltpu.make_async_copy(gid_hbm.at[src], idxg_v.at[j], sem_i).wait()

    @pl.when(s == 15)
    def _():
        pltpu.make_async_copy(h0_hbm.at[pl.ds(r0, VALID_LAST),
                                        pl.ds(col0, D_HALF)],
                              rows_v.at[pl.ds(0, VALID_LAST)], sem_h).wait()
        for j in range(3):
            src = pl.ds(r0 + IDXW * j, IDXW)
            pltpu.make_async_copy(bid_hbm.at[src], idxb_v.at[j], sem_i).wait()
            pltpu.make_async_copy(gid_hbm.at[src], idxg_v.at[j], sem_i).wait()
        tail = pl.ds(r0 + IDXW * 3, VALID_LAST - IDXW * 3)
        pltpu.make_async_copy(bid_hbm.at[tail], idxb_v.at[3, pl.ds(0, 16)],
                              sem_i).wait()
        pltpu.make_async_copy(gid_hbm.at[tail], idxg_v.at[3, pl.ds(0, 16)],
                              sem_i).wait()

    descs = []
    for j in range(GROUPS_PER_SUB):
        rows = rows_v.at[pl.ds(IDXW * j, IDXW)]
        descs.append(pltpu.async_copy(rows, blk_acc.at[idxb_v.at[j]],
                                      sem_h, add=True))
        descs.append(pltpu.async_copy(rows, gr_acc.at[idxg_v.at[j]],
                                      sem_h, add=True))
    for d in descs:
        d.wait()

    plsc.subcore_barrier()

    @pl.when(s < 15)
    def _():
        pltpu.sync_copy(blk_acc.at[pl.ds(s * 32, 32)],
                        blk_out.at[pl.ds(s * 32, 32), pl.ds(col0, D_HALF)])

    @pl.when(s == 15)
    def _():
        pltpu.sync_copy(blk_acc.at[pl.ds(480, N_BLOCKS - 480)],
                        blk_out.at[pl.ds(480, N_BLOCKS - 480),
                                   pl.ds(col0, D_HALF)])

    @pl.when(s == 0)
    def _():
        pltpu.sync_copy(gr_acc.at[pl.ds(0, N_GRAPHS)],
                        gr_out.at[:, pl.ds(col0, D_HALF)])


@jax.jit
def _segsum_sc(h0, bid, gid):
    mesh = plsc.VectorSubcoreMesh(core_axis_name="c", subcore_axis_name="s")
    f = pl.kernel(
        _sc_body,
        out_type=(
            jax.ShapeDtypeStruct((N_BLOCKS, D_FEAT), jnp.float32),
            jax.ShapeDtypeStruct((N_GRAPHS, D_FEAT), jnp.float32),
        ),
        mesh=mesh,
        compiler_params=pltpu.CompilerParams(use_tc_tiling_on_sc=False),
        scratch_types=[
            pltpu.VMEM((ROWS_PER_SUB, D_HALF), jnp.float32),
            pltpu.VMEM((GROUPS_PER_SUB, IDXW), jnp.int32),
            pltpu.VMEM((GROUPS_PER_SUB, IDXW), jnp.int32),
            pltpu.VMEM((32, D_HALF), jnp.float32),
            pltpu.VMEM_SHARED((N_BLOCKS_PAD, D_HALF), jnp.float32),
            pltpu.VMEM_SHARED((N_GRAPHS_PAD, D_HALF), jnp.float32),
            pltpu.SemaphoreType.DMA,
            pltpu.SemaphoreType.DMA,
        ],
    )
    return f(h0, bid, gid)


def kernel(H_0, Z, block_id, batch_id, edges, edge_attr):
    block_repr, graph_repr = _segsum_sc(H_0, block_id.astype(jnp.int32),
                                        batch_id.astype(jnp.int32))
    return (H_0, block_repr, graph_repr, Z)


# speedup vs baseline: 5.6538x; 1.0646x over previous
import jax
import jax.numpy as jnp
from jax import lax
from jax.experimental import pallas as pl
from jax.experimental.pallas import tpu as pltpu
from jax.experimental.pallas import tpu_sc as plsc

N_NODES = 10000
D_FEAT = 128
N_BLOCKS = 500
N_BLOCKS_PAD = 512
N_GRAPHS = 16

IDXW = 128
GROUPS_PER_SUB = 5
ROWS_PER_SUB = IDXW * GROUPS_PER_SUB
VALID_LAST = N_NODES - 15 * ROWS_PER_SUB

D_HALF = D_FEAT // 2

TC_CHUNK = 2000


def _sc_body(h0_hbm, bid_hbm, blk_out,
             rows_v, idxb_v, zero_v, blk_acc, sems, sem_i):
    c = lax.axis_index("c")
    s = lax.axis_index("s")
    col0 = c * D_HALF
    r0 = s * ROWS_PER_SUB

    @pl.when(s < 15)
    def _():
        for j in range(GROUPS_PER_SUB):
            pltpu.async_copy(
                h0_hbm.at[pl.ds(r0 + IDXW * j, IDXW), pl.ds(col0, D_HALF)],
                rows_v.at[pl.ds(IDXW * j, IDXW)], sems.at[j])
            pltpu.async_copy(bid_hbm.at[pl.ds(r0 + IDXW * j, IDXW)],
                             idxb_v.at[j], sem_i)

    @pl.when(s == 15)
    def _():
        for j in range(3):
            pltpu.async_copy(
                h0_hbm.at[pl.ds(r0 + IDXW * j, IDXW), pl.ds(col0, D_HALF)],
                rows_v.at[pl.ds(IDXW * j, IDXW)], sems.at[j])
            pltpu.async_copy(bid_hbm.at[pl.ds(r0 + IDXW * j, IDXW)],
                             idxb_v.at[j], sem_i)
        nt = VALID_LAST - IDXW * 3
        pltpu.async_copy(h0_hbm.at[pl.ds(r0 + IDXW * 3, nt),
                                   pl.ds(col0, D_HALF)],
                         rows_v.at[pl.ds(IDXW * 3, nt)], sems.at[3])
        pltpu.async_copy(bid_hbm.at[pl.ds(r0 + IDXW * 3, nt)],
                         idxb_v.at[3, pl.ds(0, nt)], sem_i)

    z16 = jnp.zeros((16,), jnp.float32)
    for i in range(32):
        for j in range(4):
            zero_v[i, pl.ds(16 * j, 16)] = z16

    @pl.when(s == 15)
    def _():
        dump = jnp.full((16,), N_BLOCKS_PAD - 1, jnp.int32)
        for k in range(1, 8):
            idxb_v[3, pl.ds(16 * k, 16)] = dump
        for k in range(8):
            idxb_v[4, pl.ds(16 * k, 16)] = dump

    pltpu.sync_copy(zero_v, blk_acc.at[pl.ds(s * 32, 32)])

    plsc.subcore_barrier()

    @pl.when(s < 15)
    def _():
        for j in range(GROUPS_PER_SUB):
            pltpu.make_async_copy(bid_hbm.at[pl.ds(r0 + IDXW * j, IDXW)],
                                  idxb_v.at[j], sem_i).wait()

    @pl.when(s == 15)
    def _():
        for j in range(3):
            pltpu.make_async_copy(bid_hbm.at[pl.ds(r0 + IDXW * j, IDXW)],
                                  idxb_v.at[j], sem_i).wait()
        nt = VALID_LAST - IDXW * 3
        pltpu.make_async_copy(bid_hbm.at[pl.ds(r0 + IDXW * 3, nt)],
                              idxb_v.at[3, pl.ds(0, nt)], sem_i).wait()

    descs = []
    for j in range(GROUPS_PER_SUB):
        rows = rows_v.at[pl.ds(IDXW * j, IDXW)]

        @pl.when(s < 15)
        def _(j=j, rows=rows):
            pltpu.make_async_copy(
                h0_hbm.at[pl.ds(r0 + IDXW * j, IDXW), pl.ds(col0, D_HALF)],
                rows, sems.at[j]).wait()

        if j <= 2:
            @pl.when(s == 15)
            def _(j=j, rows=rows):
                pltpu.make_async_copy(
                    h0_hbm.at[pl.ds(r0 + IDXW * j, IDXW),
                              pl.ds(col0, D_HALF)],
                    rows, sems.at[j]).wait()
        elif j == 3:
            @pl.when(s == 15)
            def _(j=j):
                nt = VALID_LAST - IDXW * 3
                pltpu.make_async_copy(
                    h0_hbm.at[pl.ds(r0 + IDXW * 3, nt), pl.ds(col0, D_HALF)],
                    rows_v.at[pl.ds(IDXW * 3, nt)], sems.at[3]).wait()

        descs.append(pltpu.async_copy(rows, blk_acc.at[idxb_v.at[j]],
                                      sem_i, add=True))
    for d in descs:
        d.wait()

    plsc.subcore_barrier()

    @pl.when(s < 15)
    def _():
        pltpu.sync_copy(blk_acc.at[pl.ds(s * 32, 32)],
                        blk_out.at[pl.ds(s * 32, 32), pl.ds(col0, D_HALF)])

    @pl.when(s == 15)
    def _():
        pltpu.sync_copy(blk_acc.at[pl.ds(480, N_BLOCKS - 480)],
                        blk_out.at[pl.ds(480, N_BLOCKS - 480),
                                   pl.ds(col0, D_HALF)])


def _tc_body(gid_ref, x_ref, out_ref):
    ids = gid_ref[...].reshape(N_NODES, 1)
    onehot = (ids == lax.broadcasted_iota(jnp.int32, (1, N_GRAPHS), 1)
              ).astype(jnp.float32)
    out_ref[...] = lax.dot_general(onehot, x_ref[...],
                                   (((0,), (0,)), ((), ())),
                                   preferred_element_type=jnp.float32)


@jax.jit
def _encoder_pool(h0, bid, gid):
    mesh = plsc.VectorSubcoreMesh(core_axis_name="c", subcore_axis_name="s")
    sc = pl.kernel(
        _sc_body,
        out_type=jax.ShapeDtypeStruct((N_BLOCKS, D_FEAT), jnp.float32),
        mesh=mesh,
        compiler_params=pltpu.CompilerParams(use_tc_tiling_on_sc=False),
        scratch_types=[
            pltpu.VMEM((ROWS_PER_SUB, D_HALF), jnp.float32),
            pltpu.VMEM((GROUPS_PER_SUB, IDXW), jnp.int32),
            pltpu.VMEM((32, D_HALF), jnp.float32),
            pltpu.VMEM_SHARED((N_BLOCKS_PAD, D_HALF), jnp.float32),
            pltpu.SemaphoreType.DMA((GROUPS_PER_SUB,)),
            pltpu.SemaphoreType.DMA,
        ],
    )
    block_repr = sc(h0, bid)

    graph_repr = pl.pallas_call(
        _tc_body,
        out_shape=jax.ShapeDtypeStruct((N_GRAPHS, D_FEAT), jnp.float32),
    )(gid, h0)

    return block_repr, graph_repr


def kernel(H_0, Z, block_id, batch_id, edges, edge_attr):
    block_repr, graph_repr = _encoder_pool(
        H_0, block_id.astype(jnp.int32), batch_id.astype(jnp.int32))
    return (H_0, block_repr, graph_repr, Z)
